# Initial kernel scaffold; baseline (speedup 1.0000x reference)
import jax
import jax.numpy as jnp
from jax.experimental import pallas as pl


def kernel(q, kv, indices):
    raise NotImplementedError("write your pallas kernel here")


# speedup vs baseline: 1.2171x; 1.2171x over previous
import functools
import math

import jax
import jax.numpy as jnp
from jax import lax
from jax.experimental import pallas as pl
from jax.experimental.pallas import tpu as pltpu
from jax.experimental.pallas import tpu_sc as plsc

B, S, H, SKV, G, D, T, K = 32, 1, 16, 8192, 1, 128, 64, 1024
DT = D + T

CHUNK = 128
N_CHUNK = K // CHUNK


def _sc_gather(kv_flat, idx_flat):
    info = plsc.get_sparse_core_info()
    nc = info.num_cores

    mesh = plsc.VectorSubcoreMesh(core_axis_name="c", subcore_axis_name="s")

    @functools.partial(
        pl.kernel,
        mesh=mesh,
        compiler_params=pltpu.CompilerParams(use_tc_tiling_on_sc=False),
        out_type=jax.ShapeDtypeStruct((B * K, DT), jnp.float32),
        scratch_types=[
            pltpu.VMEM((K,), jnp.int32),
            pltpu.VMEM((CHUNK, DT), jnp.float32),
            pltpu.SemaphoreType.DMA,
        ],
    )
    def gather_kernel(kv_hbm, idx_hbm, out_hbm, idx_v, rows_v, gsem):
        wid = lax.axis_index("s") * nc + lax.axis_index("c")
        base = wid * K
        pltpu.sync_copy(idx_hbm.at[pl.ds(base, K)], idx_v)

        off = wid * SKV

        def add_off(i, _):
            sl = pl.ds(i * 16, 16)
            idx_v[sl] = idx_v[sl] + off
            return 0

        lax.fori_loop(0, K // 16, add_off, 0)

        def chunk(j, _):
            pltpu.async_copy(
                kv_hbm.at[idx_v.at[pl.ds(j * CHUNK, CHUNK)]],
                rows_v, gsem).wait()
            pltpu.sync_copy(rows_v, out_hbm.at[pl.ds(base + j * CHUNK, CHUNK)])
            return 0

        lax.fori_loop(0, N_CHUNK, chunk, 0)

    return gather_kernel(kv_flat, idx_flat)


def _attn_body(q_ref, g_ref, o_ref):
    sm_scale = 1.0 / math.sqrt(DT)
    qb = q_ref[0]
    g = g_ref[0]
    logits = lax.dot_general(
        qb, g, (((1,), (1,)), ((), ())),
        preferred_element_type=jnp.float32) * sm_scale
    m = jnp.max(logits, axis=1, keepdims=True)
    p = jnp.exp(logits - m)
    s = jnp.sum(p, axis=1, keepdims=True)
    o = lax.dot_general(
        p, g[:, :D], (((1,), (0,)), ((), ())),
        preferred_element_type=jnp.float32)
    o_ref[0] = o / s


def kernel(q, kv, indices):
    kv_flat = kv.reshape(B * SKV, DT)
    idx_flat = indices.reshape(B * K)
    gathered = _sc_gather(kv_flat, idx_flat).reshape(B, K, DT)

    out = pl.pallas_call(
        _attn_body,
        grid=(B,),
        in_specs=[
            pl.BlockSpec((1, H, DT), lambda b: (b, 0, 0)),
            pl.BlockSpec((1, K, DT), lambda b: (b, 0, 0)),
        ],
        out_specs=pl.BlockSpec((1, H, D), lambda b: (b, 0, 0)),
        out_shape=jax.ShapeDtypeStruct((B, H, D), jnp.float32),
    )(q.reshape(B, H, DT), gathered)
    return out.reshape(B, S, H, D)


# speedup vs baseline: 6.8603x; 5.6364x over previous
import functools
import math

import jax
import jax.numpy as jnp
from jax import lax
from jax.experimental import pallas as pl
from jax.experimental.pallas import tpu as pltpu
from jax.experimental.pallas import tpu_sc as plsc

B, S, H, SKV, G, D, T, K = 32, 1, 16, 8192, 1, 128, 64, 1024
DT = D + T


def _sc_counts(idx_flat):
    info = plsc.get_sparse_core_info()
    nc = info.num_cores

    mesh = plsc.VectorSubcoreMesh(core_axis_name="c", subcore_axis_name="s")

    @functools.partial(
        pl.kernel,
        mesh=mesh,
        compiler_params=pltpu.CompilerParams(needs_layout_passes=False),
        out_type=jax.ShapeDtypeStruct((B, SKV), jnp.float32),
        scratch_types=[
            pltpu.VMEM((K,), jnp.int32),
            pltpu.VMEM((SKV,), jnp.float32),
        ],
    )
    def counts_kernel(idx_hbm, out_hbm, idx_v, cnt_v):
        wid = lax.axis_index("s") * nc + lax.axis_index("c")
        pltpu.sync_copy(idx_hbm.at[pl.ds(wid * K, K)], idx_v)

        zeros = jnp.zeros((16,), jnp.float32)

        def zero_body(i, _):
            cnt_v[pl.ds(i * 16, 16)] = zeros
            return 0

        lax.fori_loop(0, SKV // 16, zero_body, 0)

        ones = jnp.ones((16,), jnp.float32)

        def acc_body(i, _):
            idx16 = idx_v[pl.ds(i * 16, 16)]
            plsc.addupdate_scatter(cnt_v, [idx16], ones)
            return 0

        lax.fori_loop(0, K // 16, acc_body, 0)

        pltpu.sync_copy(cnt_v, out_hbm.at[wid])

    return counts_kernel(idx_flat)


def _attn_body(q_ref, kvt_ref, cnt_ref, o_ref):
    sm_scale = 1.0 / math.sqrt(DT)
    qb = q_ref[0]
    kvt = kvt_ref[0]
    c = cnt_ref[0]
    logits = lax.dot_general(
        qb, kvt, (((1,), (0,)), ((), ())),
        preferred_element_type=jnp.float32) * sm_scale
    bias = jnp.where(c > 0.0, jnp.log(c), -jnp.inf)
    logits = logits + bias
    m = jnp.max(logits, axis=1, keepdims=True)
    p = jnp.exp(logits - m)
    s = jnp.sum(p, axis=1, keepdims=True)
    o = lax.dot_general(
        p, kvt[:D, :], (((1,), (1,)), ((), ())),
        preferred_element_type=jnp.float32)
    o_ref[0] = o / s


def kernel(q, kv, indices):
    idx_flat = indices.reshape(B * K)
    counts = _sc_counts(idx_flat).reshape(B, 1, SKV)

    kvt = jnp.transpose(kv, (0, 3, 2, 1)).reshape(B, DT, SKV)

    out = pl.pallas_call(
        _attn_body,
        grid=(B,),
        in_specs=[
            pl.BlockSpec((1, H, DT), lambda b: (b, 0, 0)),
            pl.BlockSpec((1, DT, SKV), lambda b: (b, 0, 0)),
            pl.BlockSpec((1, 1, SKV), lambda b: (b, 0, 0)),
        ],
        out_specs=pl.BlockSpec((1, H, D), lambda b: (b, 0, 0)),
        out_shape=jax.ShapeDtypeStruct((B, H, D), jnp.float32),
    )(q.reshape(B, H, DT), kvt, counts)
    return out.reshape(B, S, H, D)
